# baseline (speedup 1.0000x reference)
import functools

import jax
import jax.numpy as jnp
from jax import lax
from jax.experimental import pallas as pl
from jax.experimental.pallas import tpu as pltpu
from jax.experimental.pallas import tpu_sc as plsc

NC = 2
NS = 16
NW = NC * NS
LANES = 16
NB = 4
CW = 8
ROWG = 640
SUBG = ROWG // 128
TCB = 3200


def _tc_pack_body(feat_ref, coords_ref, pack_ref, keys_ref):
    f = feat_ref[...]
    c = coords_ref[...]
    b = c[:, 0:1]
    z = c[:, 1:2]
    y = c[:, 2:3]
    x = c[:, 3:4]
    cols = [f, b,
            (x + 0.5) * 0.05 + 0.0,
            (y + 0.5) * 0.05 + (-40.0),
            (z + 0.5) * 0.1 + (-3.0)]
    if CW > 8:
        cols.append(jnp.zeros((TCB, CW - 8), jnp.float32))
    pack_ref[...] = jnp.concatenate(cols, axis=1)
    keys_ref[...] = c[:, 0].astype(jnp.int32).reshape(1, TCB // 128, 128)


def _tc_split_body(pack_ref, feat_ref, coords_ref):
    p = pack_ref[...]
    feat_ref[...] = p[:, 0:4]
    coords_ref[...] = p[:, 4:8]


def _sc_body(n_groups, max_groups,
             pack_hbm, keys_hbm, out_pack,
             kv, pk_v, idx_v, pack_v16, all_cnt_v, shared_cnt, sem):
    cid = lax.axis_index("c")
    sid = lax.axis_index("s")
    wid = sid * NC + cid
    iota = lax.iota(jnp.int32, LANES)
    zeros16 = jnp.zeros((LANES,), jnp.int32)

    gb = n_groups // NW
    gr = n_groups % NW

    def chunk_bounds(w):
        start = w * gb + jnp.minimum(w, gr)
        cnt = gb + jnp.where(w < gr, 1, 0)
        return start, start + cnt

    def count_chunk(w):
        s0, s1 = chunk_bounds(w)

        def load_group(g, _):
            pltpu.sync_copy(keys_hbm.at[pl.ds(g * ROWG, ROWG)],
                            kv.at[pl.ds((g - s0) * ROWG, ROWG)])
            return 0
        lax.fori_loop(s0, s1, load_group, 0)

        def cstep(i, cnts):
            keys = kv[pl.ds(i * LANES, LANES)]
            return tuple(cnts[b] + (keys == b).astype(jnp.int32)
                         for b in range(NB))

        n_steps = (s1 - s0) * (ROWG // LANES)
        cnts = lax.fori_loop(0, n_steps, cstep, (zeros16,) * NB)
        packed = zeros16
        for b in range(NB):
            packed = jnp.where(iota == b, jnp.sum(cnts[b]), packed)
        pack_v16[...] = packed
        pltpu.sync_copy(pack_v16, shared_cnt.at[pl.ds(w * LANES, LANES)])

    count_chunk(sid * NC + (1 - cid))
    count_chunk(wid)
    plsc.subcore_barrier()
    pltpu.sync_copy(shared_cnt, all_cnt_v)

    def acc_step(w, carry):
        tot, pre = carry
        v = all_cnt_v[pl.ds(w * LANES, LANES)]
        pre = pre + jnp.where(w < wid, v, 0)
        return tot + v, pre

    tot, pre = lax.fori_loop(0, NW, acc_step, (zeros16, zeros16))
    off_vec = (plsc.cumsum(tot) - tot) + pre
    offs = tuple(zeros16 + jnp.sum(jnp.where(iota == b, off_vec, 0))
                 for b in range(NB))

    s0, s1 = chunk_bounds(wid)

    def do_group(gl, offs):
        g = s0 + gl
        pltpu.sync_copy(pack_hbm.at[pl.ds(g * ROWG, ROWG)], pk_v)

        copies = []
        for j in range(SUBG):
            def pstep(i2, offs, j=j):
                base = gl * ROWG + (j * 8 + i2) * LANES
                keys = kv[pl.ds(base, LANES)]
                dest = zeros16
                new_offs = []
                for b in range(NB):
                    m = keys == b
                    pc = plsc.cumsum(m.astype(jnp.int32))
                    dest = jnp.where(m, offs[b] + pc - 1, dest)
                    new_offs.append(
                        offs[b] + plsc.all_reduce_population_count(m))
                idx_v[j, pl.ds(i2 * LANES, LANES)] = dest
                return tuple(new_offs)

            offs = lax.fori_loop(0, 128 // LANES, pstep, offs)
            copies.append(pltpu.async_copy(
                pk_v.at[pl.ds(j * 128, 128)], out_pack.at[idx_v.at[j]],
                sem))
        for cp in copies:
            cp.wait()
        return offs

    lax.fori_loop(0, s1 - s0, do_group, offs)


@functools.lru_cache(maxsize=None)
def _build(n):
    assert n % ROWG == 0 and n % TCB == 0, n
    n_groups = n // ROWG
    max_groups = n_groups // NW + (1 if n_groups % NW else 0)

    tc_pack = pl.pallas_call(
        _tc_pack_body,
        grid=(n // TCB,),
        in_specs=[pl.BlockSpec((TCB, 4), lambda i: (i, 0)),
                  pl.BlockSpec((TCB, 4), lambda i: (i, 0))],
        out_specs=[pl.BlockSpec((TCB, CW), lambda i: (i, 0)),
                   pl.BlockSpec((1, TCB // 128, 128), lambda i: (i, 0, 0))],
        out_shape=[jax.ShapeDtypeStruct((n, CW), jnp.float32),
                   jax.ShapeDtypeStruct((n // TCB, TCB // 128, 128),
                                        jnp.int32)],
        name="vfec_pack_tc",
    )

    tc_split = pl.pallas_call(
        _tc_split_body,
        grid=(n // TCB,),
        in_specs=[pl.BlockSpec((TCB, CW), lambda i: (i, 0))],
        out_specs=[pl.BlockSpec((TCB, 4), lambda i: (i, 0)),
                   pl.BlockSpec((TCB, 4), lambda i: (i, 0))],
        out_shape=[jax.ShapeDtypeStruct((n, 4), jnp.float32),
                   jax.ShapeDtypeStruct((n, 4), jnp.float32)],
        name="vfec_split_tc",
    )

    mesh = plsc.VectorSubcoreMesh(core_axis_name="c", subcore_axis_name="s",
                                  num_cores=NC, num_subcores=NS)
    sc_sort = pl.kernel(
        functools.partial(_sc_body, n_groups, max_groups),
        out_type=jax.ShapeDtypeStruct((n, CW), jnp.float32),
        mesh=mesh,
        scratch_types=[
            pltpu.VMEM((max_groups * ROWG,), jnp.int32),
            pltpu.VMEM((ROWG, CW), jnp.float32),
            pltpu.VMEM((SUBG, 128), jnp.int32),
            pltpu.VMEM((LANES,), jnp.int32),
            pltpu.VMEM((NW * LANES,), jnp.int32),
            pltpu.VMEM_SHARED((NW * LANES,), jnp.int32),
            pltpu.SemaphoreType.DMA,
        ],
        name="vfec_counting_sort_sc",
        compiler_params=pltpu.CompilerParams(needs_layout_passes=False,
                                             use_tc_tiling_on_sc=False),
    )

    def run(voxel_features, voxel_coords):
        packed, keys2d = tc_pack(voxel_features, voxel_coords)
        sorted_pack = sc_sort(packed, keys2d.reshape(-1))
        return tc_split(sorted_pack)

    return run


def kernel(voxel_features, voxel_coords, batch_size):
    del batch_size
    n = voxel_features.shape[0]
    fn = _build(int(n))
    return fn(voxel_features, voxel_coords)


# speedup vs baseline: 1.0203x; 1.0203x over previous
import functools

import jax
import jax.numpy as jnp
from jax import lax
from jax.experimental import pallas as pl
from jax.experimental.pallas import tpu as pltpu
from jax.experimental.pallas import tpu_sc as plsc

NC = 2
NS = 16
NW = NC * NS
LANES = 16
NB = 4
CW = 8
ROWG = 3200
SUBG = ROWG // 128
TCB = 3200


def _tc_pack_body(feat_ref, coords_ref, pack_ref, keys_ref):
    f = feat_ref[...]
    c = coords_ref[...]
    b = c[:, 0:1]
    z = c[:, 1:2]
    y = c[:, 2:3]
    x = c[:, 3:4]
    cols = [f, b,
            (x + 0.5) * 0.05 + 0.0,
            (y + 0.5) * 0.05 + (-40.0),
            (z + 0.5) * 0.1 + (-3.0)]
    if CW > 8:
        cols.append(jnp.zeros((TCB, CW - 8), jnp.float32))
    pack_ref[...] = jnp.concatenate(cols, axis=1)
    keys_ref[...] = c[:, 0].astype(jnp.int32).reshape(1, TCB // 128, 128)


def _tc_split_body(pack_ref, feat_ref, coords_ref):
    p = pack_ref[...]
    feat_ref[...] = p[:, 0:4]
    coords_ref[...] = p[:, 4:8]


def _sc_body(n_groups, max_groups,
             pack_hbm, keys_hbm, out_pack,
             kv, pk_v, idx_v, pack_v16, all_cnt_v, shared_cnt, sem):
    cid = lax.axis_index("c")
    sid = lax.axis_index("s")
    wid = sid * NC + cid
    iota = lax.iota(jnp.int32, LANES)
    zeros16 = jnp.zeros((LANES,), jnp.int32)

    gb = n_groups // NW
    gr = n_groups % NW

    def chunk_bounds(w):
        start = w * gb + jnp.minimum(w, gr)
        cnt = gb + jnp.where(w < gr, 1, 0)
        return start, start + cnt

    def count_chunk(w):
        s0, s1 = chunk_bounds(w)

        def load_group(g, _):
            pltpu.sync_copy(keys_hbm.at[pl.ds(g * ROWG, ROWG)],
                            kv.at[pl.ds((g - s0) * ROWG, ROWG)])
            return 0
        lax.fori_loop(s0, s1, load_group, 0)

        def cstep(i, cnts):
            keys = kv[pl.ds(i * LANES, LANES)]
            return tuple(cnts[b] + (keys == b).astype(jnp.int32)
                         for b in range(NB))

        n_steps = (s1 - s0) * (ROWG // LANES)
        cnts = lax.fori_loop(0, n_steps, cstep, (zeros16,) * NB)
        packed = zeros16
        for b in range(NB):
            packed = jnp.where(iota == b, jnp.sum(cnts[b]), packed)
        pack_v16[...] = packed
        pltpu.sync_copy(pack_v16, shared_cnt.at[pl.ds(w * LANES, LANES)])

    count_chunk(sid * NC + (1 - cid))
    count_chunk(wid)
    plsc.subcore_barrier()
    pltpu.sync_copy(shared_cnt, all_cnt_v)

    def acc_step(w, carry):
        tot, pre = carry
        v = all_cnt_v[pl.ds(w * LANES, LANES)]
        pre = pre + jnp.where(w < wid, v, 0)
        return tot + v, pre

    tot, pre = lax.fori_loop(0, NW, acc_step, (zeros16, zeros16))
    off_vec = (plsc.cumsum(tot) - tot) + pre
    offs = tuple(zeros16 + jnp.sum(jnp.where(iota == b, off_vec, 0))
                 for b in range(NB))

    s0, s1 = chunk_bounds(wid)

    def do_group(gl, offs):
        g = s0 + gl
        pltpu.sync_copy(pack_hbm.at[pl.ds(g * ROWG, ROWG)], pk_v)

        copies = []
        for j in range(SUBG):
            def pstep(i2, offs, j=j):
                base = gl * ROWG + (j * 8 + i2) * LANES
                keys = kv[pl.ds(base, LANES)]
                dest = zeros16
                new_offs = []
                for b in range(NB):
                    m = keys == b
                    pc = plsc.cumsum(m.astype(jnp.int32))
                    dest = jnp.where(m, offs[b] + pc - 1, dest)
                    new_offs.append(
                        offs[b] + plsc.all_reduce_population_count(m))
                idx_v[j, pl.ds(i2 * LANES, LANES)] = dest
                return tuple(new_offs)

            offs = lax.fori_loop(0, 128 // LANES, pstep, offs)
            copies.append(pltpu.async_copy(
                pk_v.at[pl.ds(j * 128, 128)], out_pack.at[idx_v.at[j]],
                sem))
        for cp in copies:
            cp.wait()
        return offs

    lax.fori_loop(0, s1 - s0, do_group, offs)


@functools.lru_cache(maxsize=None)
def _build(n):
    assert n % ROWG == 0 and n % TCB == 0, n
    n_groups = n // ROWG
    max_groups = n_groups // NW + (1 if n_groups % NW else 0)

    tc_pack = pl.pallas_call(
        _tc_pack_body,
        grid=(n // TCB,),
        in_specs=[pl.BlockSpec((TCB, 4), lambda i: (i, 0)),
                  pl.BlockSpec((TCB, 4), lambda i: (i, 0))],
        out_specs=[pl.BlockSpec((TCB, CW), lambda i: (i, 0)),
                   pl.BlockSpec((1, TCB // 128, 128), lambda i: (i, 0, 0))],
        out_shape=[jax.ShapeDtypeStruct((n, CW), jnp.float32),
                   jax.ShapeDtypeStruct((n // TCB, TCB // 128, 128),
                                        jnp.int32)],
        name="vfec_pack_tc",
    )

    tc_split = pl.pallas_call(
        _tc_split_body,
        grid=(n // TCB,),
        in_specs=[pl.BlockSpec((TCB, CW), lambda i: (i, 0))],
        out_specs=[pl.BlockSpec((TCB, 4), lambda i: (i, 0)),
                   pl.BlockSpec((TCB, 4), lambda i: (i, 0))],
        out_shape=[jax.ShapeDtypeStruct((n, 4), jnp.float32),
                   jax.ShapeDtypeStruct((n, 4), jnp.float32)],
        name="vfec_split_tc",
    )

    mesh = plsc.VectorSubcoreMesh(core_axis_name="c", subcore_axis_name="s",
                                  num_cores=NC, num_subcores=NS)
    sc_sort = pl.kernel(
        functools.partial(_sc_body, n_groups, max_groups),
        out_type=jax.ShapeDtypeStruct((n, CW), jnp.float32),
        mesh=mesh,
        scratch_types=[
            pltpu.VMEM((max_groups * ROWG,), jnp.int32),
            pltpu.VMEM((ROWG, CW), jnp.float32),
            pltpu.VMEM((SUBG, 128), jnp.int32),
            pltpu.VMEM((LANES,), jnp.int32),
            pltpu.VMEM((NW * LANES,), jnp.int32),
            pltpu.VMEM_SHARED((NW * LANES,), jnp.int32),
            pltpu.SemaphoreType.DMA,
        ],
        name="vfec_counting_sort_sc",
        compiler_params=pltpu.CompilerParams(needs_layout_passes=False,
                                             use_tc_tiling_on_sc=False),
    )

    def run(voxel_features, voxel_coords):
        packed, keys2d = tc_pack(voxel_features, voxel_coords)
        sorted_pack = sc_sort(packed, keys2d.reshape(-1))
        return tc_split(sorted_pack)

    return run


def kernel(voxel_features, voxel_coords, batch_size):
    del batch_size
    n = voxel_features.shape[0]
    fn = _build(int(n))
    return fn(voxel_features, voxel_coords)
